# Initial kernel scaffold; baseline (speedup 1.0000x reference)
import jax
import jax.numpy as jnp
from jax.experimental import pallas as pl


def kernel(x, edge_index, batch, W1, b1, eps1, W2, b2, eps2, W3, b3, eps3):
    raise NotImplementedError("write your pallas kernel here")


# speedup vs baseline: 7.4828x; 7.4828x over previous
import functools

import jax
import jax.numpy as jnp
from jax import lax
from jax.experimental import pallas as pl
from jax.experimental.pallas import tpu as pltpu
from jax.experimental.pallas import tpu_sc as plsc

N = 10000
E = 320000
D_IN = 128
H = 256
NC = 2
NS = 16

EPT = E // NS
WIN = 80
N_PAD = 10240
NPT = N_PAD // NS
WB = 128
NWB = NPT // WB


def _make_agg(dh, edge_split, interpret=False):
    ept = E // (2 * NS) if edge_split else E // NS
    nw = ept // WIN
    mesh = plsc.VectorSubcoreMesh(
        core_axis_name="c", subcore_axis_name="s",
        num_cores=NC, num_subcores=NS)

    @functools.partial(
        pl.kernel,
        out_type=jax.ShapeDtypeStruct((2 * N_PAD, dh), jnp.float32),
        mesh=mesh,
        interpret=interpret,
        scratch_types=[
            pltpu.VMEM((WIN,), jnp.int32),
            pltpu.VMEM((WIN,), jnp.int32),
            pltpu.VMEM((WIN,), jnp.int32),
            pltpu.VMEM((WIN,), jnp.int32),
            pltpu.VMEM((WIN, dh), jnp.float32),
            pltpu.VMEM((WIN, dh), jnp.float32),
            pltpu.VMEM((WB, dh), jnp.float32),
            pltpu.VMEM_SHARED((N_PAD, dh), jnp.float32),
            pltpu.SemaphoreType.DMA,
            pltpu.SemaphoreType.DMA,
            pltpu.SemaphoreType.DMA,
            pltpu.SemaphoreType.DMA,
        ],
    )
    def agg_kernel(h_hbm, src_hbm, dst_hbm, zeros_hbm, agg_hbm,
                   src0, dst0, src1, dst1, rows0, rows1,
                   stage_v, acc_sh, si0, si1, sg0, sg1):
        c = lax.axis_index("c")
        s = lax.axis_index("s")
        srcs, dsts, rows = [src0, src1], [dst0, dst1], [rows0, rows1]
        sis, sgs = [si0, si1], [sg0, sg1]

        pltpu.sync_copy(zeros_hbm, stage_v)

        def zero_body(k, carry):
            pltpu.sync_copy(stage_v, acc_sh.at[pl.ds(s * NPT + k * WB, WB)])
            return carry
        lax.fori_loop(0, NWB, zero_body, 0)
        plsc.subcore_barrier()

        if edge_split:
            dbase = c * (E // 2) + s * ept
            sbase = dbase
        else:
            dbase = s * ept
            sbase = c * E + dbase

        def idx_issue(b, w):
            pltpu.async_copy(src_hbm.at[pl.ds(sbase + w * WIN, WIN)],
                             srcs[b], sis[b])
            pltpu.async_copy(dst_hbm.at[pl.ds(dbase + w * WIN, WIN)],
                             dsts[b], sis[b])

        def idx_wait(b, w):
            pltpu.make_async_copy(src_hbm.at[pl.ds(sbase + w * WIN, WIN)],
                                  srcs[b], sis[b]).wait()
            pltpu.make_async_copy(dst_hbm.at[pl.ds(dbase + w * WIN, WIN)],
                                  dsts[b], sis[b]).wait()

        def g_issue(b):
            pltpu.async_copy(h_hbm.at[srcs[b]], rows[b], sgs[b])

        def g_wait(b):
            pltpu.make_async_copy(h_hbm.at[srcs[b]], rows[b], sgs[b]).wait()

        def scat(b):
            pltpu.sync_copy(rows[b], acc_sh.at[dsts[b]], add=True)

        idx_issue(0, 0)
        idx_issue(1, 1)
        idx_wait(0, 0)
        g_issue(0)

        peel = 2 + (nw % 2)

        def pair_body(gi, carry):
            for b in (0, 1):
                w = 2 * gi + b
                g_wait(b)
                idx_wait(1 - b, w + 1)
                g_issue(1 - b)
                scat(b)
                idx_issue(b, w + 2)
            return carry
        lax.fori_loop(0, (nw - peel) // 2, pair_body, 0)

        for wp in range(nw - peel, nw):
            b = wp % 2
            g_wait(b)
            if wp + 1 < nw:
                idx_wait(1 - b, wp + 1)
                g_issue(1 - b)
            scat(b)
            if wp + 2 < nw:
                idx_issue(b, wp + 2)
        plsc.subcore_barrier()

        def out_body(k, carry):
            r0 = s * NPT + k * WB
            pltpu.sync_copy(acc_sh.at[pl.ds(r0, WB)], stage_v)
            pltpu.sync_copy(stage_v, agg_hbm.at[pl.ds(c * N_PAD + r0, WB)])
            return carry
        lax.fori_loop(0, NWB, out_body, 0)

    return agg_kernel


BR = 2000


def _mlp1_body(scale_ref, x_ref, a_ref, w_ref, b_ref, o_ref):
    hin = scale_ref[0] * x_ref[...] + a_ref[0] + a_ref[1]
    o = jnp.dot(hin, w_ref[...], preferred_element_type=jnp.float32)
    o_ref[0] = jnp.maximum(o + b_ref[...], 0.0)


def _make_mlp1(interpret=False):
    return pl.pallas_call(
        _mlp1_body,
        grid=(N // BR, 2),
        in_specs=[
            pl.BlockSpec(memory_space=pltpu.SMEM),
            pl.BlockSpec((BR, D_IN), lambda i, j: (i, 0)),
            pl.BlockSpec((2, BR, D_IN), lambda i, j: (0, i, 0)),
            pl.BlockSpec((D_IN, H // 2), lambda i, j: (0, j)),
            pl.BlockSpec((1, H // 2), lambda i, j: (0, j)),
        ],
        out_specs=pl.BlockSpec((1, BR, H // 2), lambda i, j: (j, i, 0)),
        out_shape=jax.ShapeDtypeStruct((2, N, H // 2), jnp.float32),
        interpret=interpret,
    )


def _mlp_body(scale_ref, x_ref, a_ref, w_ref, b_ref, o_ref):
    xf = jnp.concatenate([x_ref[0], x_ref[1]], axis=1)
    af = jnp.concatenate([a_ref[0], a_ref[1]], axis=1)
    hin = scale_ref[0] * xf + af
    o = jnp.dot(hin, w_ref[...], preferred_element_type=jnp.float32)
    o_ref[0] = jnp.maximum(o + b_ref[...], 0.0)


def _make_mlp(dh, d, interpret=False):
    return pl.pallas_call(
        _mlp_body,
        grid=(N // BR, 2),
        in_specs=[
            pl.BlockSpec(memory_space=pltpu.SMEM),
            pl.BlockSpec((2, BR, dh), lambda i, j: (0, i, 0)),
            pl.BlockSpec((2, BR, dh), lambda i, j: (0, i, 0)),
            pl.BlockSpec((d, H // 2), lambda i, j: (0, j)),
            pl.BlockSpec((1, H // 2), lambda i, j: (0, j)),
        ],
        out_specs=pl.BlockSpec((1, BR, H // 2), lambda i, j: (j, i, 0)),
        out_shape=jax.ShapeDtypeStruct((2, N, H // 2), jnp.float32),
        interpret=interpret,
    )


def _lsm_body(scale_ref, x_ref, a_ref, w_ref, b_ref, o_ref):
    xf = jnp.concatenate([x_ref[0], x_ref[1]], axis=1)
    af = jnp.concatenate([a_ref[0], a_ref[1]], axis=1)
    hin = scale_ref[0] * xf + af
    z = jnp.dot(hin, w_ref[...], preferred_element_type=jnp.float32)
    z = jnp.maximum(z + b_ref[...], 0.0)
    m = jnp.max(z, axis=1, keepdims=True)
    ze = z - m
    lse = jnp.log(jnp.sum(jnp.exp(ze), axis=1, keepdims=True))
    o_ref[...] = ze - lse


def _make_lsm(dh, d, interpret=False):
    return pl.pallas_call(
        _lsm_body,
        grid=(N // BR,),
        in_specs=[
            pl.BlockSpec(memory_space=pltpu.SMEM),
            pl.BlockSpec((2, BR, dh), lambda i: (0, i, 0)),
            pl.BlockSpec((2, BR, dh), lambda i: (0, i, 0)),
            pl.BlockSpec((d, H), lambda i: (0, 0)),
            pl.BlockSpec((1, H), lambda i: (0, 0)),
        ],
        out_specs=pl.BlockSpec((BR, H), lambda i: (i, 0)),
        out_shape=jax.ShapeDtypeStruct((N, H), jnp.float32),
        interpret=interpret,
    )


_make_agg = functools.lru_cache(maxsize=None)(_make_agg)
_make_mlp1 = functools.lru_cache(maxsize=None)(_make_mlp1)
_make_mlp = functools.lru_cache(maxsize=None)(_make_mlp)
_make_lsm = functools.lru_cache(maxsize=None)(_make_lsm)


def kernel(x, edge_index, batch, W1, b1, eps1, W2, b2, eps2, W3, b3, eps3):
    src = edge_index[0]
    dst = edge_index[1]
    srcoff = jnp.concatenate([src, src + N])
    zeros128 = jnp.zeros((WB, 128), jnp.float32)

    s1 = (1.0 + eps1).reshape(1)
    s2 = (1.0 + eps2).reshape(1)
    s3 = (1.0 + eps3).reshape(1)

    agg1 = _make_agg(128, True)(x, src, dst, zeros128)
    h1 = _make_mlp1()(
        s1, x, agg1.reshape(2, N_PAD, 128), W1, b1.reshape(1, H))

    agg2 = _make_agg(128, False)(h1.reshape(2 * N, 128), srcoff, dst, zeros128)
    h2 = _make_mlp(H // 2, H)(
        s2, h1, agg2.reshape(2, N_PAD, 128), W2, b2.reshape(1, H))

    agg3 = _make_agg(128, False)(h2.reshape(2 * N, 128), srcoff, dst, zeros128)
    out = _make_lsm(H // 2, H)(
        s3, h2, agg3.reshape(2, N_PAD, 128), W3, b3.reshape(1, H))
    return out


# speedup vs baseline: 9.9965x; 1.3359x over previous
import functools

import jax
import jax.numpy as jnp
from jax import lax
from jax.experimental import pallas as pl
from jax.experimental.pallas import tpu as pltpu
from jax.experimental.pallas import tpu_sc as plsc

N = 10000
E = 320000
D_IN = 128
H = 256
NC = 2
NS = 16

EPT = E // NS
WIN = 80
N_PAD = 10240
NPT = N_PAD // NS
WB = 80
NWB = NPT // WB


def _make_agg(dh, edge_split, interpret=False):
    ept = E // (2 * NS) if edge_split else E // NS
    nw = ept // WIN
    mesh = plsc.VectorSubcoreMesh(
        core_axis_name="c", subcore_axis_name="s",
        num_cores=NC, num_subcores=NS)

    idx_scratch = [pltpu.VMEM((WIN,), jnp.int32) for _ in range(16)]
    row_scratch = [pltpu.VMEM((WIN, dh), jnp.float32) for _ in range(4)]
    sem_scratch = [pltpu.SemaphoreType.DMA for _ in range(12)]

    @functools.partial(
        pl.kernel,
        out_type=jax.ShapeDtypeStruct((2 * N_PAD, dh), jnp.float32),
        mesh=mesh,
        interpret=interpret,
        scratch_types=idx_scratch + row_scratch + [
            pltpu.VMEM_SHARED((N_PAD, dh), jnp.float32),
        ] + sem_scratch,
    )
    def agg_kernel(h_hbm, src_hbm, dst_hbm, zeros_hbm, agg_hbm, *refs):
        srcs = list(refs[0:8])
        dsts = list(refs[8:16])
        rows = list(refs[16:20])
        acc_sh = refs[20]
        sis = list(refs[21:25])
        sgs = list(refs[25:29])
        sss = list(refs[29:33])
        stage_v = rows[0]
        c = lax.axis_index("c")
        s = lax.axis_index("s")

        pltpu.sync_copy(zeros_hbm, stage_v)

        def zero_body(k, carry):
            pltpu.sync_copy(stage_v, acc_sh.at[pl.ds(s * NPT + k * WB, WB)])
            return carry
        lax.fori_loop(0, NWB, zero_body, 0)
        plsc.subcore_barrier()

        if edge_split:
            dbase = c * (E // 2) + s * ept
            sbase = dbase
        else:
            dbase = s * ept
            sbase = c * E + dbase

        def idx_issue(w, q):
            pltpu.async_copy(src_hbm.at[pl.ds(sbase + w * WIN, WIN)],
                             srcs[q], sis[q % 4])
            pltpu.async_copy(dst_hbm.at[pl.ds(dbase + w * WIN, WIN)],
                             dsts[q], sis[q % 4])

        def idx_wait(w, q):
            pltpu.make_async_copy(src_hbm.at[pl.ds(sbase + w * WIN, WIN)],
                                  srcs[q], sis[q % 4]).wait()
            pltpu.make_async_copy(dst_hbm.at[pl.ds(dbase + w * WIN, WIN)],
                                  dsts[q], sis[q % 4]).wait()

        def g_issue(b, q):
            pltpu.async_copy(h_hbm.at[srcs[q]], rows[b], sgs[b])

        def g_wait(b, q):
            pltpu.make_async_copy(h_hbm.at[srcs[q]], rows[b], sgs[b]).wait()

        def s_issue(b, q):
            pltpu.async_copy(rows[b], acc_sh.at[dsts[q]], sss[b], add=True)

        def s_wait(b, q):
            pltpu.make_async_copy(rows[b], acc_sh.at[dsts[q]], sss[b]).wait()

        def win_step(w, b, q, has_prev2, has_next2, has_next4):
            g_wait(b, q)
            s_issue(b, q)
            if has_prev2:
                s_wait((b + 2) % 4, (q + 6) % 8)
            if has_next2:
                idx_wait(w + 2, (q + 2) % 8)
                g_issue((b + 2) % 4, (q + 2) % 8)
            if has_next4:
                idx_issue(w + 4, (q + 4) % 8)

        for wq in range(4):
            idx_issue(wq, wq)
        idx_wait(0, 0)
        g_issue(0, 0)
        idx_wait(1, 1)
        g_issue(1, 1)

        for wp in range(2):
            win_step(wp, wp % 4, wp % 8, False, wp + 2 < nw, wp + 4 < nw)

        L = max(((nw - 6) // 8) * 8, 0)

        def main_body(gi, carry):
            for u in range(8):
                w = 2 + 8 * gi + u
                win_step(w, (2 + u) % 4, (2 + u) % 8, True, True, True)
            return carry
        lax.fori_loop(0, L // 8, main_body, 0)

        for wp in range(2 + L, nw):
            win_step(wp, wp % 4, wp % 8, True, wp + 2 < nw, wp + 4 < nw)

        s_wait((nw - 2) % 4, (nw - 2) % 8)
        s_wait((nw - 1) % 4, (nw - 1) % 8)
        plsc.subcore_barrier()

        def out_body(k, carry):
            r0 = s * NPT + k * WB
            pltpu.sync_copy(acc_sh.at[pl.ds(r0, WB)], stage_v)
            pltpu.sync_copy(stage_v, agg_hbm.at[pl.ds(c * N_PAD + r0, WB)])
            return carry
        lax.fori_loop(0, NWB, out_body, 0)

    return agg_kernel


BR = 2000


def _mlp1_body(scale_ref, x_ref, a_ref, w_ref, b_ref, o_ref):
    hin = scale_ref[0] * x_ref[...] + a_ref[0] + a_ref[1]
    o = jnp.dot(hin, w_ref[...], preferred_element_type=jnp.float32)
    o_ref[0] = jnp.maximum(o + b_ref[...], 0.0)


def _make_mlp1(interpret=False):
    return pl.pallas_call(
        _mlp1_body,
        grid=(N // BR, 2),
        in_specs=[
            pl.BlockSpec(memory_space=pltpu.SMEM),
            pl.BlockSpec((BR, D_IN), lambda i, j: (i, 0)),
            pl.BlockSpec((2, BR, D_IN), lambda i, j: (0, i, 0)),
            pl.BlockSpec((D_IN, H // 2), lambda i, j: (0, j)),
            pl.BlockSpec((1, H // 2), lambda i, j: (0, j)),
        ],
        out_specs=pl.BlockSpec((1, BR, H // 2), lambda i, j: (j, i, 0)),
        out_shape=jax.ShapeDtypeStruct((2, N, H // 2), jnp.float32),
        interpret=interpret,
    )


def _mlp_body(scale_ref, x_ref, a_ref, w_ref, b_ref, o_ref):
    xf = jnp.concatenate([x_ref[0], x_ref[1]], axis=1)
    af = jnp.concatenate([a_ref[0], a_ref[1]], axis=1)
    hin = scale_ref[0] * xf + af
    o = jnp.dot(hin, w_ref[...], preferred_element_type=jnp.float32)
    o_ref[0] = jnp.maximum(o + b_ref[...], 0.0)


def _make_mlp(dh, d, interpret=False):
    return pl.pallas_call(
        _mlp_body,
        grid=(N // BR, 2),
        in_specs=[
            pl.BlockSpec(memory_space=pltpu.SMEM),
            pl.BlockSpec((2, BR, dh), lambda i, j: (0, i, 0)),
            pl.BlockSpec((2, BR, dh), lambda i, j: (0, i, 0)),
            pl.BlockSpec((d, H // 2), lambda i, j: (0, j)),
            pl.BlockSpec((1, H // 2), lambda i, j: (0, j)),
        ],
        out_specs=pl.BlockSpec((1, BR, H // 2), lambda i, j: (j, i, 0)),
        out_shape=jax.ShapeDtypeStruct((2, N, H // 2), jnp.float32),
        interpret=interpret,
    )


def _lsm_body(scale_ref, x_ref, a_ref, w_ref, b_ref, o_ref):
    xf = jnp.concatenate([x_ref[0], x_ref[1]], axis=1)
    af = jnp.concatenate([a_ref[0], a_ref[1]], axis=1)
    hin = scale_ref[0] * xf + af
    z = jnp.dot(hin, w_ref[...], preferred_element_type=jnp.float32)
    z = jnp.maximum(z + b_ref[...], 0.0)
    m = jnp.max(z, axis=1, keepdims=True)
    ze = z - m
    lse = jnp.log(jnp.sum(jnp.exp(ze), axis=1, keepdims=True))
    o_ref[...] = ze - lse


def _make_lsm(dh, d, interpret=False):
    return pl.pallas_call(
        _lsm_body,
        grid=(N // BR,),
        in_specs=[
            pl.BlockSpec(memory_space=pltpu.SMEM),
            pl.BlockSpec((2, BR, dh), lambda i: (0, i, 0)),
            pl.BlockSpec((2, BR, dh), lambda i: (0, i, 0)),
            pl.BlockSpec((d, H), lambda i: (0, 0)),
            pl.BlockSpec((1, H), lambda i: (0, 0)),
        ],
        out_specs=pl.BlockSpec((BR, H), lambda i: (i, 0)),
        out_shape=jax.ShapeDtypeStruct((N, H), jnp.float32),
        interpret=interpret,
    )


_make_agg = functools.lru_cache(maxsize=None)(_make_agg)
_make_mlp1 = functools.lru_cache(maxsize=None)(_make_mlp1)
_make_mlp = functools.lru_cache(maxsize=None)(_make_mlp)
_make_lsm = functools.lru_cache(maxsize=None)(_make_lsm)


def kernel(x, edge_index, batch, W1, b1, eps1, W2, b2, eps2, W3, b3, eps3):
    src = edge_index[0]
    dst = edge_index[1]
    srcoff = jnp.concatenate([src, src + N])
    zeros128 = jnp.zeros((WB, 128), jnp.float32)

    s1 = (1.0 + eps1).reshape(1)
    s2 = (1.0 + eps2).reshape(1)
    s3 = (1.0 + eps3).reshape(1)

    agg1 = _make_agg(128, True)(x, src, dst, zeros128)
    h1 = _make_mlp1()(
        s1, x, agg1.reshape(2, N_PAD, 128), W1, b1.reshape(1, H))

    agg2 = _make_agg(128, False)(h1.reshape(2 * N, 128), srcoff, dst, zeros128)
    h2 = _make_mlp(H // 2, H)(
        s2, h1, agg2.reshape(2, N_PAD, 128), W2, b2.reshape(1, H))

    agg3 = _make_agg(128, False)(h2.reshape(2 * N, 128), srcoff, dst, zeros128)
    out = _make_lsm(H // 2, H)(
        s3, h2, agg3.reshape(2, N_PAD, 128), W3, b3.reshape(1, H))
    return out


# speedup vs baseline: 11.1006x; 1.1104x over previous
import functools

import jax
import jax.numpy as jnp
from jax import lax
from jax.experimental import pallas as pl
from jax.experimental.pallas import tpu as pltpu
from jax.experimental.pallas import tpu_sc as plsc

N = 10000
E = 320000
D_IN = 128
H = 256
NC = 2
NS = 16

WIN = 120
SEG = E // 32
SEGP = 10080
EP = 32 * SEGP
N_PAD = 10240
NPT = N_PAD // NS
WB = 80
NWB = NPT // WB


def _make_agg(dh, edge_split, interpret=False):
    nw = (SEGP // WIN) if edge_split else (2 * SEGP // WIN)
    mesh = plsc.VectorSubcoreMesh(
        core_axis_name="c", subcore_axis_name="s",
        num_cores=NC, num_subcores=NS)

    idx_scratch = [pltpu.VMEM((WIN,), jnp.int32) for _ in range(12)]
    row_scratch = [pltpu.VMEM((WIN, dh), jnp.float32) for _ in range(3)]
    sem_scratch = [pltpu.SemaphoreType.DMA for _ in range(9)]

    @functools.partial(
        pl.kernel,
        out_type=jax.ShapeDtypeStruct((2 * N_PAD, dh), jnp.float32),
        mesh=mesh,
        interpret=interpret,
        scratch_types=idx_scratch + row_scratch + [
            pltpu.VMEM_SHARED((N_PAD, dh), jnp.float32),
        ] + sem_scratch,
    )
    def agg_kernel(h_hbm, src_hbm, dst_hbm, zeros_hbm, agg_hbm, *refs):
        srcs = list(refs[0:6])
        dsts = list(refs[6:12])
        rows = list(refs[12:15])
        acc_sh = refs[15]
        sis = list(refs[16:19])
        sgs = list(refs[19:22])
        sss = list(refs[22:25])
        stage_v = rows[0].at[pl.ds(0, WB)]
        c = lax.axis_index("c")
        s = lax.axis_index("s")

        pltpu.sync_copy(zeros_hbm, stage_v)

        def zero_body(k, carry):
            pltpu.sync_copy(stage_v, acc_sh.at[pl.ds(s * NPT + k * WB, WB)])
            return carry
        lax.fori_loop(0, NWB, zero_body, 0)
        plsc.subcore_barrier()

        if edge_split:
            dbase = (c * NS + s) * SEGP
            sbase = dbase
        else:
            dbase = s * 2 * SEGP
            sbase = c * EP + dbase

        def idx_issue(w, q):
            pltpu.async_copy(src_hbm.at[pl.ds(sbase + w * WIN, WIN)],
                             srcs[q], sis[q % 3])
            pltpu.async_copy(dst_hbm.at[pl.ds(dbase + w * WIN, WIN)],
                             dsts[q], sis[q % 3])

        def idx_wait(w, q):
            pltpu.make_async_copy(src_hbm.at[pl.ds(sbase + w * WIN, WIN)],
                                  srcs[q], sis[q % 3]).wait()
            pltpu.make_async_copy(dst_hbm.at[pl.ds(dbase + w * WIN, WIN)],
                                  dsts[q], sis[q % 3]).wait()

        def g_issue(b, q):
            pltpu.async_copy(h_hbm.at[srcs[q]], rows[b], sgs[b])

        def g_wait(b, q):
            pltpu.make_async_copy(h_hbm.at[srcs[q]], rows[b], sgs[b]).wait()

        def s_issue(b, q):
            pltpu.async_copy(rows[b], acc_sh.at[dsts[q]], sss[b], add=True)

        def s_wait(b, q):
            pltpu.make_async_copy(rows[b], acc_sh.at[dsts[q]], sss[b]).wait()

        def win_step(w, b, q, has_prev, has_next2, has_next4):
            g_wait(b, q)
            s_issue(b, q)
            if has_prev:
                s_wait((b + 2) % 3, (q + 5) % 6)
            if has_next2:
                idx_wait(w + 2, (q + 2) % 6)
                g_issue((b + 2) % 3, (q + 2) % 6)
            if has_next4:
                idx_issue(w + 4, (q + 4) % 6)

        for wq in range(4):
            idx_issue(wq, wq)
        idx_wait(0, 0)
        g_issue(0, 0)
        idx_wait(1, 1)
        g_issue(1, 1)

        win_step(0, 0, 0, False, True, True)
        win_step(1, 1, 1, True, True, True)

        L = max(((nw - 6) // 6) * 6, 0)

        def main_body(gi, carry):
            for u in range(6):
                w = 2 + 6 * gi + u
                win_step(w, (2 + u) % 3, (2 + u) % 6, True, True, True)
            return carry
        lax.fori_loop(0, L // 6, main_body, 0)

        for wp in range(2 + L, nw):
            win_step(wp, wp % 3, wp % 6, True, wp + 2 < nw, wp + 4 < nw)

        s_wait((nw - 1) % 3, (nw - 1) % 6)
        plsc.subcore_barrier()

        def out_body(k, carry):
            r0 = s * NPT + k * WB
            pltpu.sync_copy(acc_sh.at[pl.ds(r0, WB)], stage_v)
            pltpu.sync_copy(stage_v, agg_hbm.at[pl.ds(c * N_PAD + r0, WB)])
            return carry
        lax.fori_loop(0, NWB, out_body, 0)

    return agg_kernel


BR = 2000


def _mlp1_body(scale_ref, x_ref, a_ref, w_ref, b_ref, o_ref):
    hin = scale_ref[0] * x_ref[...] + a_ref[0] + a_ref[1]
    o = jnp.dot(hin, w_ref[...], preferred_element_type=jnp.float32)
    o_ref[0] = jnp.maximum(o + b_ref[...], 0.0)


def _make_mlp1(interpret=False):
    return pl.pallas_call(
        _mlp1_body,
        grid=(N // BR, 2),
        in_specs=[
            pl.BlockSpec(memory_space=pltpu.SMEM),
            pl.BlockSpec((BR, D_IN), lambda i, j: (i, 0)),
            pl.BlockSpec((2, BR, D_IN), lambda i, j: (0, i, 0)),
            pl.BlockSpec((D_IN, H // 2), lambda i, j: (0, j)),
            pl.BlockSpec((1, H // 2), lambda i, j: (0, j)),
        ],
        out_specs=pl.BlockSpec((1, BR, H // 2), lambda i, j: (j, i, 0)),
        out_shape=jax.ShapeDtypeStruct((2, N, H // 2), jnp.float32),
        interpret=interpret,
    )


def _mlp_body(scale_ref, x_ref, a_ref, w_ref, b_ref, o_ref):
    xf = jnp.concatenate([x_ref[0], x_ref[1]], axis=1)
    af = jnp.concatenate([a_ref[0], a_ref[1]], axis=1)
    hin = scale_ref[0] * xf + af
    o = jnp.dot(hin, w_ref[...], preferred_element_type=jnp.float32)
    o_ref[0] = jnp.maximum(o + b_ref[...], 0.0)


def _make_mlp(dh, d, interpret=False):
    return pl.pallas_call(
        _mlp_body,
        grid=(N // BR, 2),
        in_specs=[
            pl.BlockSpec(memory_space=pltpu.SMEM),
            pl.BlockSpec((2, BR, dh), lambda i, j: (0, i, 0)),
            pl.BlockSpec((2, BR, dh), lambda i, j: (0, i, 0)),
            pl.BlockSpec((d, H // 2), lambda i, j: (0, j)),
            pl.BlockSpec((1, H // 2), lambda i, j: (0, j)),
        ],
        out_specs=pl.BlockSpec((1, BR, H // 2), lambda i, j: (j, i, 0)),
        out_shape=jax.ShapeDtypeStruct((2, N, H // 2), jnp.float32),
        interpret=interpret,
    )


def _lsm_body(scale_ref, x_ref, a_ref, w_ref, b_ref, o_ref):
    xf = jnp.concatenate([x_ref[0], x_ref[1]], axis=1)
    af = jnp.concatenate([a_ref[0], a_ref[1]], axis=1)
    hin = scale_ref[0] * xf + af
    z = jnp.dot(hin, w_ref[...], preferred_element_type=jnp.float32)
    z = jnp.maximum(z + b_ref[...], 0.0)
    m = jnp.max(z, axis=1, keepdims=True)
    ze = z - m
    lse = jnp.log(jnp.sum(jnp.exp(ze), axis=1, keepdims=True))
    o_ref[...] = ze - lse


def _make_lsm(dh, d, interpret=False):
    return pl.pallas_call(
        _lsm_body,
        grid=(N // BR,),
        in_specs=[
            pl.BlockSpec(memory_space=pltpu.SMEM),
            pl.BlockSpec((2, BR, dh), lambda i: (0, i, 0)),
            pl.BlockSpec((2, BR, dh), lambda i: (0, i, 0)),
            pl.BlockSpec((d, H), lambda i: (0, 0)),
            pl.BlockSpec((1, H), lambda i: (0, 0)),
        ],
        out_specs=pl.BlockSpec((BR, H), lambda i: (i, 0)),
        out_shape=jax.ShapeDtypeStruct((N, H), jnp.float32),
        interpret=interpret,
    )


_make_agg = functools.lru_cache(maxsize=None)(_make_agg)
_make_mlp1 = functools.lru_cache(maxsize=None)(_make_mlp1)
_make_mlp = functools.lru_cache(maxsize=None)(_make_mlp)
_make_lsm = functools.lru_cache(maxsize=None)(_make_lsm)


def kernel(x, edge_index, batch, W1, b1, eps1, W2, b2, eps2, W3, b3, eps3):
    src = edge_index[0]
    dst = edge_index[1]
    npad = SEGP - SEG
    pad_src = (jnp.arange(npad, dtype=jnp.int32) * 125) % N
    pad_dst = N + (jnp.arange(npad, dtype=jnp.int32) % (N_PAD - N))
    src_pad = jnp.concatenate(
        [src.reshape(32, SEG),
         jnp.broadcast_to(pad_src, (32, npad))], axis=1).reshape(-1)
    dst = jnp.concatenate(
        [dst.reshape(32, SEG),
         jnp.broadcast_to(pad_dst, (32, npad))], axis=1).reshape(-1)
    src = src_pad
    srcoff = jnp.concatenate([src, src + N])
    zeros128 = jnp.zeros((WB, 128), jnp.float32)

    s1 = (1.0 + eps1).reshape(1)
    s2 = (1.0 + eps2).reshape(1)
    s3 = (1.0 + eps3).reshape(1)

    agg1 = _make_agg(128, True)(x, src, dst, zeros128)
    h1 = _make_mlp1()(
        s1, x, agg1.reshape(2, N_PAD, 128), W1, b1.reshape(1, H))

    agg2 = _make_agg(128, False)(h1.reshape(2 * N, 128), srcoff, dst, zeros128)
    h2 = _make_mlp(H // 2, H)(
        s2, h1, agg2.reshape(2, N_PAD, 128), W2, b2.reshape(1, H))

    agg3 = _make_agg(128, False)(h2.reshape(2 * N, 128), srcoff, dst, zeros128)
    out = _make_lsm(H // 2, H)(
        s3, h2, agg3.reshape(2, N_PAD, 128), W3, b3.reshape(1, H))
    return out


# speedup vs baseline: 11.2061x; 1.0095x over previous
import functools

import jax
import jax.numpy as jnp
from jax import lax
from jax.experimental import pallas as pl
from jax.experimental.pallas import tpu as pltpu
from jax.experimental.pallas import tpu_sc as plsc

N = 10000
E = 320000
D_IN = 128
H = 256
NC = 2
NS = 16

WIN = 120
SEG = E // 32
SEGP = 10080
EP = 32 * SEGP
N_PAD = 10240
NPT = N_PAD // NS
WB = 80
NWB = NPT // WB


def _make_agg(dh, edge_split, interpret=False):
    nw = (SEGP // WIN) if edge_split else (2 * SEGP // WIN)
    mesh = plsc.VectorSubcoreMesh(
        core_axis_name="c", subcore_axis_name="s",
        num_cores=NC, num_subcores=NS)

    idx_scratch = [pltpu.VMEM((WIN,), jnp.int32) for _ in range(12)]
    row_scratch = [pltpu.VMEM((WIN, dh), jnp.float32) for _ in range(3)]
    sem_scratch = [pltpu.SemaphoreType.DMA for _ in range(9)]

    @functools.partial(
        pl.kernel,
        out_type=jax.ShapeDtypeStruct((2 * N_PAD, dh), jnp.float32),
        mesh=mesh,
        interpret=interpret,
        scratch_types=idx_scratch + row_scratch + [
            pltpu.VMEM_SHARED((N_PAD, dh), jnp.float32),
        ] + sem_scratch,
    )
    def agg_kernel(h_hbm, src_hbm, dst_hbm, zeros_hbm, agg_hbm, *refs):
        srcs = list(refs[0:6])
        dsts = list(refs[6:12])
        rows = list(refs[12:15])
        acc_sh = refs[15]
        sis = list(refs[16:19])
        sgs = list(refs[19:22])
        sss = list(refs[22:25])
        c = lax.axis_index("c")
        s = lax.axis_index("s")

        if edge_split:
            dbase = (c * NS + s) * SEGP
            sbase = dbase
        else:
            dbase = s * 2 * SEGP
            sbase = c * EP + dbase

        def idx_issue(w, q):
            pltpu.async_copy(src_hbm.at[pl.ds(sbase + w * WIN, WIN)],
                             srcs[q], sis[q % 3])
            pltpu.async_copy(dst_hbm.at[pl.ds(dbase + w * WIN, WIN)],
                             dsts[q], sis[q % 3])

        def idx_wait(w, q):
            pltpu.make_async_copy(src_hbm.at[pl.ds(sbase + w * WIN, WIN)],
                                  srcs[q], sis[q % 3]).wait()
            pltpu.make_async_copy(dst_hbm.at[pl.ds(dbase + w * WIN, WIN)],
                                  dsts[q], sis[q % 3]).wait()

        def g_issue(b, q):
            pltpu.async_copy(h_hbm.at[srcs[q]], rows[b], sgs[b])

        def g_wait(b, q):
            pltpu.make_async_copy(h_hbm.at[srcs[q]], rows[b], sgs[b]).wait()

        def s_issue(b, q):
            pltpu.async_copy(rows[b], acc_sh.at[dsts[q]], sss[b], add=True)

        def s_wait(b, q):
            pltpu.make_async_copy(rows[b], acc_sh.at[dsts[q]], sss[b]).wait()

        def win_step(w, b, q, has_prev, has_next2, has_next4):
            g_wait(b, q)
            s_issue(b, q)
            if has_prev:
                s_wait((b + 2) % 3, (q + 5) % 6)
            if has_next2:
                idx_wait(w + 2, (q + 2) % 6)
                g_issue((b + 2) % 3, (q + 2) % 6)
            if has_next4:
                idx_issue(w + 4, (q + 4) % 6)

        for wq in range(4):
            idx_issue(wq, wq)
        idx_wait(0, 0)
        g_issue(0, 0)
        idx_wait(1, 1)
        g_issue(1, 1)

        stage_z = rows[2].at[pl.ds(0, WB)]
        pltpu.sync_copy(zeros_hbm, stage_z)

        def zero_body(k, carry):
            pltpu.sync_copy(stage_z, acc_sh.at[pl.ds(s * NPT + k * WB, WB)])
            return carry
        lax.fori_loop(0, NWB, zero_body, 0)
        plsc.subcore_barrier()

        win_step(0, 0, 0, False, True, True)
        win_step(1, 1, 1, True, True, True)

        L = max(((nw - 6) // 6) * 6, 0)

        def main_body(gi, carry):
            for u in range(6):
                w = 2 + 6 * gi + u
                win_step(w, (2 + u) % 3, (2 + u) % 6, True, True, True)
            return carry
        lax.fori_loop(0, L // 6, main_body, 0)

        for wp in range(2 + L, nw):
            win_step(wp, wp % 3, wp % 6, True, wp + 2 < nw, wp + 4 < nw)

        s_wait((nw - 1) % 3, (nw - 1) % 6)
        plsc.subcore_barrier()

        stages = [r.at[pl.ds(0, WB)] for r in rows]

        def wb_rd(k):
            return pltpu.make_async_copy(
                acc_sh.at[pl.ds(s * NPT + k * WB, WB)],
                stages[k % 3], sgs[k % 3])

        def wb_wr(k):
            return pltpu.make_async_copy(
                stages[k % 3],
                agg_hbm.at[pl.ds(c * N_PAD + s * NPT + k * WB, WB)],
                sss[k % 3])

        wb_rd(0).start()
        wb_rd(1).start()
        for k in range(NWB):
            wb_rd(k).wait()
            wb_wr(k).start()
            if k >= 1:
                wb_wr(k - 1).wait()
            if k + 2 < NWB:
                wb_rd(k + 2).start()
        wb_wr(NWB - 1).wait()

    return agg_kernel


BR = 2000


def _mlp1_body(scale_ref, x_ref, a_ref, w_ref, b_ref, o_ref):
    hin = scale_ref[0] * x_ref[...] + a_ref[0] + a_ref[1]
    o = jnp.dot(hin, w_ref[...], preferred_element_type=jnp.float32)
    o_ref[0] = jnp.maximum(o + b_ref[...], 0.0)


def _make_mlp1(interpret=False):
    return pl.pallas_call(
        _mlp1_body,
        grid=(N // BR, 2),
        in_specs=[
            pl.BlockSpec(memory_space=pltpu.SMEM),
            pl.BlockSpec((BR, D_IN), lambda i, j: (i, 0)),
            pl.BlockSpec((2, BR, D_IN), lambda i, j: (0, i, 0)),
            pl.BlockSpec((D_IN, H // 2), lambda i, j: (0, j)),
            pl.BlockSpec((1, H // 2), lambda i, j: (0, j)),
        ],
        out_specs=pl.BlockSpec((1, BR, H // 2), lambda i, j: (j, i, 0)),
        out_shape=jax.ShapeDtypeStruct((2, N, H // 2), jnp.float32),
        interpret=interpret,
    )


def _mlp_body(scale_ref, x_ref, a_ref, w_ref, b_ref, o_ref):
    xf = jnp.concatenate([x_ref[0], x_ref[1]], axis=1)
    af = jnp.concatenate([a_ref[0], a_ref[1]], axis=1)
    hin = scale_ref[0] * xf + af
    o = jnp.dot(hin, w_ref[...], preferred_element_type=jnp.float32)
    o_ref[0] = jnp.maximum(o + b_ref[...], 0.0)


def _make_mlp(dh, d, interpret=False):
    return pl.pallas_call(
        _mlp_body,
        grid=(N // BR, 2),
        in_specs=[
            pl.BlockSpec(memory_space=pltpu.SMEM),
            pl.BlockSpec((2, BR, dh), lambda i, j: (0, i, 0)),
            pl.BlockSpec((2, BR, dh), lambda i, j: (0, i, 0)),
            pl.BlockSpec((d, H // 2), lambda i, j: (0, j)),
            pl.BlockSpec((1, H // 2), lambda i, j: (0, j)),
        ],
        out_specs=pl.BlockSpec((1, BR, H // 2), lambda i, j: (j, i, 0)),
        out_shape=jax.ShapeDtypeStruct((2, N, H // 2), jnp.float32),
        interpret=interpret,
    )


def _lsm_body(scale_ref, x_ref, a_ref, w_ref, b_ref, o_ref):
    xf = jnp.concatenate([x_ref[0], x_ref[1]], axis=1)
    af = jnp.concatenate([a_ref[0], a_ref[1]], axis=1)
    hin = scale_ref[0] * xf + af
    z = jnp.dot(hin, w_ref[...], preferred_element_type=jnp.float32)
    z = jnp.maximum(z + b_ref[...], 0.0)
    m = jnp.max(z, axis=1, keepdims=True)
    ze = z - m
    lse = jnp.log(jnp.sum(jnp.exp(ze), axis=1, keepdims=True))
    o_ref[...] = ze - lse


def _make_lsm(dh, d, interpret=False):
    return pl.pallas_call(
        _lsm_body,
        grid=(N // BR,),
        in_specs=[
            pl.BlockSpec(memory_space=pltpu.SMEM),
            pl.BlockSpec((2, BR, dh), lambda i: (0, i, 0)),
            pl.BlockSpec((2, BR, dh), lambda i: (0, i, 0)),
            pl.BlockSpec((d, H), lambda i: (0, 0)),
            pl.BlockSpec((1, H), lambda i: (0, 0)),
        ],
        out_specs=pl.BlockSpec((BR, H), lambda i: (i, 0)),
        out_shape=jax.ShapeDtypeStruct((N, H), jnp.float32),
        interpret=interpret,
    )


_make_agg = functools.lru_cache(maxsize=None)(_make_agg)
_make_mlp1 = functools.lru_cache(maxsize=None)(_make_mlp1)
_make_mlp = functools.lru_cache(maxsize=None)(_make_mlp)
_make_lsm = functools.lru_cache(maxsize=None)(_make_lsm)


def kernel(x, edge_index, batch, W1, b1, eps1, W2, b2, eps2, W3, b3, eps3):
    src = edge_index[0]
    dst = edge_index[1]
    npad = SEGP - SEG
    pad_src = (jnp.arange(npad, dtype=jnp.int32) * 125) % N
    pad_dst = N + (jnp.arange(npad, dtype=jnp.int32) % (N_PAD - N))
    src_pad = jnp.concatenate(
        [src.reshape(32, SEG),
         jnp.broadcast_to(pad_src, (32, npad))], axis=1).reshape(-1)
    dst = jnp.concatenate(
        [dst.reshape(32, SEG),
         jnp.broadcast_to(pad_dst, (32, npad))], axis=1).reshape(-1)
    src = src_pad
    srcoff = jnp.concatenate([src, src + N])
    zeros128 = jnp.zeros((WB, 128), jnp.float32)

    s1 = (1.0 + eps1).reshape(1)
    s2 = (1.0 + eps2).reshape(1)
    s3 = (1.0 + eps3).reshape(1)

    agg1 = _make_agg(128, True)(x, src, dst, zeros128)
    h1 = _make_mlp1()(
        s1, x, agg1.reshape(2, N_PAD, 128), W1, b1.reshape(1, H))

    agg2 = _make_agg(128, False)(h1.reshape(2 * N, 128), srcoff, dst, zeros128)
    h2 = _make_mlp(H // 2, H)(
        s2, h1, agg2.reshape(2, N_PAD, 128), W2, b2.reshape(1, H))

    agg3 = _make_agg(128, False)(h2.reshape(2 * N, 128), srcoff, dst, zeros128)
    out = _make_lsm(H // 2, H)(
        s3, h2, agg3.reshape(2, N_PAD, 128), W3, b3.reshape(1, H))
    return out
